# baseline (speedup 1.0000x reference)
import functools

import jax
import jax.numpy as jnp
from jax import lax
from jax.experimental import pallas as pl
from jax.experimental.pallas import tpu as pltpu
from jax.experimental.pallas import tpu_sc as plsc

NCORES = 2
NSUB = 16
NW = NCORES * NSUB
CH = 128
LN_EPS = 1e-5


def _row_chunks(total):
    out, off = [], 0
    while off < total:
        sz = min(CH, total - off)
        out.append((off, sz))
        off += sz
    return out


def _zero_vmem(ref, rows, d, value=0.0):
    @pl.loop(0, rows)
    def _(i):
        @pl.loop(0, d // 16)
        def _(k):
            ref[i, pl.ds(k * 16, 16)] = jnp.full((16,), value, jnp.float32)


def _make_sc_agg(n, d, chunks, n_pad):
    rows_per_tile = n_pad // NSUB
    assert rows_per_tile % 8 == 0 and chunks % 8 == 0

    out_type = jax.ShapeDtypeStruct((NCORES, n_pad, d), jnp.float32)
    scratch = [
        pltpu.VMEM_SHARED((n_pad, d), jnp.float32),
        pltpu.VMEM((8, CH), jnp.int32),
        pltpu.VMEM((8, CH), jnp.int32),
        pltpu.VMEM((CH, d), jnp.float32),
        pltpu.VMEM((CH, d), jnp.float32),
        pltpu.SemaphoreType.DMA,
        pltpu.SemaphoreType.DMA,
    ]
    mesh = plsc.VectorSubcoreMesh(core_axis_name="core", subcore_axis_name="subcore")

    def body(h_hbm, src_hbm, dst_hbm, agg_out,
             agg_sp, src_v, dst_v, rows0, rows1, sem0, sem1):
        c = lax.axis_index("core")
        s = lax.axis_index("subcore")
        w = c * NSUB + s
        rows = (rows0, rows1)
        sems = (sem0, sem1)
        zbase = s * rows_per_tile

        with jax.named_scope("agg_zero"):
            _zero_vmem(rows0, CH, d)
            for off, sz in _row_chunks(rows_per_tile):
                pltpu.sync_copy(rows0.at[pl.ds(0, sz)],
                                agg_sp.at[pl.ds(zbase + off, sz)])

            plsc.subcore_barrier()

        h_c = h_hbm.at[c]

        def wait(b):
            pltpu.make_async_copy(h_c.at[pl.ds(0, CH)], rows[b], sems[b]).wait()

        with jax.named_scope("agg_edges"):
            @pl.loop(0, chunks // 8)
            def _(jb):
                pltpu.sync_copy(src_hbm.at[w].at[pl.ds(jb * 8, 8)], src_v)
                pltpu.sync_copy(dst_hbm.at[w].at[pl.ds(jb * 8, 8)], dst_v)
                pltpu.async_copy(h_c.at[src_v.at[0]], rows0, sem0)
                for t in range(4):
                    c0, c1 = 2 * t, 2 * t + 1
                    pltpu.async_copy(h_c.at[src_v.at[c1]], rows1, sem1)
                    wait(0)
                    pltpu.sync_copy(rows0, agg_sp.at[dst_v.at[c0]], add=True)
                    if c1 + 1 < 8:
                        pltpu.async_copy(h_c.at[src_v.at[c1 + 1]], rows0, sem0)
                    wait(1)
                    pltpu.sync_copy(rows1, agg_sp.at[dst_v.at[c1]], add=True)

            plsc.subcore_barrier()

        with jax.named_scope("agg_writeout"):
            for off, sz in _row_chunks(rows_per_tile):
                pltpu.sync_copy(agg_sp.at[pl.ds(zbase + off, sz)],
                                rows0.at[pl.ds(0, sz)])
                pltpu.sync_copy(rows0.at[pl.ds(0, sz)],
                                agg_out.at[c].at[pl.ds(zbase + off, sz)])

    return pl.kernel(body, out_type=out_type, mesh=mesh, scratch_types=scratch)


def _make_sc_deg(n, d, chunks, n_pad):
    rows_per_tile = n_pad // NSUB
    assert rows_per_tile % 8 == 0 and chunks % 8 == 0

    out_type = jax.ShapeDtypeStruct((NCORES, n_pad, d), jnp.float32)
    scratch = [
        pltpu.VMEM_SHARED((n_pad, d), jnp.float32),
        pltpu.VMEM((8, CH), jnp.int32),
        pltpu.VMEM((CH, d), jnp.float32),
    ]
    mesh = plsc.VectorSubcoreMesh(core_axis_name="core", subcore_axis_name="subcore")

    def body(dst_hbm, deg_out, deg_sp, dst_v, rows_v):
        c = lax.axis_index("core")
        s = lax.axis_index("subcore")
        w = c * NSUB + s
        zbase = s * rows_per_tile

        _zero_vmem(rows_v, CH, d)
        for off, sz in _row_chunks(rows_per_tile):
            pltpu.sync_copy(rows_v.at[pl.ds(0, sz)],
                            deg_sp.at[pl.ds(zbase + off, sz)])

        _zero_vmem(rows_v, CH, d, value=1.0)

        plsc.subcore_barrier()

        @pl.loop(0, chunks // 8)
        def _(jb):
            pltpu.sync_copy(dst_hbm.at[w].at[pl.ds(jb * 8, 8)], dst_v)

            @pl.loop(0, 8)
            def _(j):
                pltpu.sync_copy(rows_v, deg_sp.at[dst_v.at[j]], add=True)

        plsc.subcore_barrier()

        for off, sz in _row_chunks(rows_per_tile):
            pltpu.sync_copy(deg_sp.at[pl.ds(zbase + off, sz)],
                            rows_v.at[pl.ds(0, sz)])
            pltpu.sync_copy(rows_v.at[pl.ds(0, sz)],
                            deg_out.at[c].at[pl.ds(zbase + off, sz)])

    return pl.kernel(body, out_type=out_type, mesh=mesh, scratch_types=scratch)


def _tc_update(h, p, degp, wl, wr, bb, gg, be):
    n, d = h.shape
    r = 1000
    assert n % r == 0

    def body(p_ref, dg_ref, h_ref, wl_ref, wr_ref, b_ref, g_ref, be_ref, o_ref):
        deg = dg_ref[0, :, 0:1] + dg_ref[1, :, 0:1]
        deg = jnp.maximum(deg, 1.0)
        agg = (p_ref[0] + p_ref[1]) / deg
        hh = h_ref[...]
        t = (jnp.dot(agg, wl_ref[...], preferred_element_type=jnp.float32,
                     precision=lax.Precision.HIGHEST)
             + jnp.dot(hh, wr_ref[...], preferred_element_type=jnp.float32,
                       precision=lax.Precision.HIGHEST)
             + b_ref[...])
        mu = jnp.mean(t, axis=-1, keepdims=True)
        var = jnp.mean((t - mu) ** 2, axis=-1, keepdims=True)
        t = (t - mu) * lax.rsqrt(var + LN_EPS) * g_ref[...] + be_ref[...]
        o_ref[...] = hh + jnp.maximum(t, 0.0)

    return pl.pallas_call(
        body,
        grid=(n // r,),
        in_specs=[
            pl.BlockSpec((2, r, d), lambda i: (0, i, 0)),
            pl.BlockSpec((2, r, 16), lambda i: (0, i, 0)),
            pl.BlockSpec((r, d), lambda i: (i, 0)),
            pl.BlockSpec((d, d), lambda i: (0, 0)),
            pl.BlockSpec((d, d), lambda i: (0, 0)),
            pl.BlockSpec((1, d), lambda i: (0, 0)),
            pl.BlockSpec((1, d), lambda i: (0, 0)),
            pl.BlockSpec((1, d), lambda i: (0, 0)),
        ],
        out_specs=pl.BlockSpec((r, d), lambda i: (i, 0)),
        out_shape=jax.ShapeDtypeStruct((n, d), jnp.float32),
    )(p, degp, h, wl, wr, bb, gg, be)


def kernel(node_features, edge_index, Wl, Wr, b, gamma, beta):
    n, d = node_features.shape
    e = edge_index.shape[1]
    l = Wl.shape[0]

    chunks = -(-(-(-e // (NW * CH))) // 8) * 8
    e_pad = NW * chunks * CH
    n_pad = -(-(n + 1) // (NSUB * 8)) * (NSUB * 8)

    src = edge_index[0].astype(jnp.int32)
    dst = edge_index[1].astype(jnp.int32)
    pad = e_pad - e
    pad_dst = n + (jnp.arange(pad, dtype=jnp.int32) % (n_pad - n))
    src_r = jnp.concatenate([src, jnp.zeros((pad,), jnp.int32)]).reshape(NW, chunks, CH)
    dst_r = jnp.concatenate([dst, pad_dst]).reshape(NW, chunks, CH)

    sc_agg = _make_sc_agg(n, d, chunks, n_pad)
    sc_deg = _make_sc_deg(n, d, chunks, n_pad)

    degp = sc_deg(dst_r)[:, :, :16]

    h = node_features
    for i in range(l):
        p = sc_agg(jnp.stack([h, h]), src_r, dst_r)
        h = _tc_update(h, p, degp, Wl[i], Wr[i],
                       b[i].reshape(1, d), gamma[i].reshape(1, d),
                       beta[i].reshape(1, d))
    return h


# speedup vs baseline: 1.0419x; 1.0419x over previous
import functools

import jax
import jax.numpy as jnp
from jax import lax
from jax.experimental import pallas as pl
from jax.experimental.pallas import tpu as pltpu
from jax.experimental.pallas import tpu_sc as plsc

NCORES = 2
NSUB = 16
NW = NCORES * NSUB
CH = 128
LN_EPS = 1e-5


def _row_chunks(total):
    out, off = [], 0
    while off < total:
        sz = min(CH, total - off)
        out.append((off, sz))
        off += sz
    return out


def _zero_vmem(ref, rows, d, value=0.0):
    @pl.loop(0, rows)
    def _(i):
        @pl.loop(0, d // 16)
        def _(k):
            ref[i, pl.ds(k * 16, 16)] = jnp.full((16,), value, jnp.float32)


def _make_sc_agg(n, d, chunks, n_pad):
    rows_per_tile = n_pad // NSUB
    assert rows_per_tile % 8 == 0 and chunks % 8 == 0

    out_type = jax.ShapeDtypeStruct((NCORES, n_pad, d), jnp.float32)
    scratch = [
        pltpu.VMEM_SHARED((n_pad, d), jnp.float32),
        pltpu.VMEM((8, CH), jnp.int32),
        pltpu.VMEM((8, CH), jnp.int32),
        pltpu.VMEM((CH, d), jnp.float32),
        pltpu.VMEM((CH, d), jnp.float32),
        pltpu.SemaphoreType.DMA,
        pltpu.SemaphoreType.DMA,
    ]
    mesh = plsc.VectorSubcoreMesh(core_axis_name="core", subcore_axis_name="subcore")

    def body(h_hbm, src_hbm, dst_hbm, agg_out,
             agg_sp, src_v, dst_v, rows0, rows1, sem0, sem1):
        c = lax.axis_index("core")
        s = lax.axis_index("subcore")
        w = c * NSUB + s
        rows = (rows0, rows1)
        sems = (sem0, sem1)
        zbase = s * rows_per_tile

        with jax.named_scope("agg_zero"):
            _zero_vmem(rows0, CH, d)
            for off, sz in _row_chunks(rows_per_tile):
                pltpu.sync_copy(rows0.at[pl.ds(0, sz)],
                                agg_sp.at[pl.ds(zbase + off, sz)])

            plsc.subcore_barrier()

        def gather(j_ref, b):
            pltpu.async_copy(h_hbm.at[j_ref.at[pl.ds(0, 64)]],
                             rows[b].at[pl.ds(0, 64)], sems[b])
            pltpu.async_copy(h_hbm.at[j_ref.at[pl.ds(64, 64)]],
                             rows[b].at[pl.ds(64, 64)], sems[b])

        def wait(b):
            pltpu.make_async_copy(h_hbm.at[pl.ds(0, CH)], rows[b], sems[b]).wait()

        with jax.named_scope("agg_edges"):
            @pl.loop(0, chunks // 8)
            def _(jb):
                pltpu.sync_copy(src_hbm.at[w].at[pl.ds(jb * 8, 8)], src_v)
                pltpu.sync_copy(dst_hbm.at[w].at[pl.ds(jb * 8, 8)], dst_v)
                gather(src_v.at[0], 0)
                for t in range(4):
                    c0, c1 = 2 * t, 2 * t + 1
                    gather(src_v.at[c1], 1)
                    wait(0)
                    pltpu.sync_copy(rows0, agg_sp.at[dst_v.at[c0]], add=True)
                    if c1 + 1 < 8:
                        gather(src_v.at[c1 + 1], 0)
                    wait(1)
                    pltpu.sync_copy(rows1, agg_sp.at[dst_v.at[c1]], add=True)

            plsc.subcore_barrier()

        with jax.named_scope("agg_writeout"):
            for off, sz in _row_chunks(rows_per_tile):
                pltpu.sync_copy(agg_sp.at[pl.ds(zbase + off, sz)],
                                rows0.at[pl.ds(0, sz)])
                pltpu.sync_copy(rows0.at[pl.ds(0, sz)],
                                agg_out.at[c].at[pl.ds(zbase + off, sz)])

    return pl.kernel(body, out_type=out_type, mesh=mesh, scratch_types=scratch)


def _make_sc_deg(n, d, chunks, n_pad):
    rows_per_tile = n_pad // NSUB
    assert rows_per_tile % 8 == 0 and chunks % 8 == 0

    out_type = jax.ShapeDtypeStruct((NCORES, n_pad, d), jnp.float32)
    scratch = [
        pltpu.VMEM_SHARED((n_pad, d), jnp.float32),
        pltpu.VMEM((8, CH), jnp.int32),
        pltpu.VMEM((CH, d), jnp.float32),
    ]
    mesh = plsc.VectorSubcoreMesh(core_axis_name="core", subcore_axis_name="subcore")

    def body(dst_hbm, deg_out, deg_sp, dst_v, rows_v):
        c = lax.axis_index("core")
        s = lax.axis_index("subcore")
        w = c * NSUB + s
        zbase = s * rows_per_tile

        _zero_vmem(rows_v, CH, d)
        for off, sz in _row_chunks(rows_per_tile):
            pltpu.sync_copy(rows_v.at[pl.ds(0, sz)],
                            deg_sp.at[pl.ds(zbase + off, sz)])

        _zero_vmem(rows_v, CH, d, value=1.0)

        plsc.subcore_barrier()

        @pl.loop(0, chunks // 8)
        def _(jb):
            pltpu.sync_copy(dst_hbm.at[w].at[pl.ds(jb * 8, 8)], dst_v)

            @pl.loop(0, 8)
            def _(j):
                pltpu.sync_copy(rows_v, deg_sp.at[dst_v.at[j]], add=True)

        plsc.subcore_barrier()

        for off, sz in _row_chunks(rows_per_tile):
            pltpu.sync_copy(deg_sp.at[pl.ds(zbase + off, sz)],
                            rows_v.at[pl.ds(0, sz)])
            pltpu.sync_copy(rows_v.at[pl.ds(0, sz)],
                            deg_out.at[c].at[pl.ds(zbase + off, sz)])

    return pl.kernel(body, out_type=out_type, mesh=mesh, scratch_types=scratch)


def _tc_update(h, p, degp, wl, wr, bb, gg, be):
    n, d = h.shape
    r = 1000
    assert n % r == 0

    def body(p_ref, dg_ref, h_ref, wl_ref, wr_ref, b_ref, g_ref, be_ref, o_ref):
        deg = dg_ref[0, :, 0:1] + dg_ref[1, :, 0:1]
        deg = jnp.maximum(deg, 1.0)
        agg = (p_ref[0] + p_ref[1]) / deg
        hh = h_ref[...]
        t = (jnp.dot(agg, wl_ref[...], preferred_element_type=jnp.float32,
                     precision=lax.Precision.HIGHEST)
             + jnp.dot(hh, wr_ref[...], preferred_element_type=jnp.float32,
                       precision=lax.Precision.HIGHEST)
             + b_ref[...])
        mu = jnp.mean(t, axis=-1, keepdims=True)
        var = jnp.mean((t - mu) ** 2, axis=-1, keepdims=True)
        t = (t - mu) * lax.rsqrt(var + LN_EPS) * g_ref[...] + be_ref[...]
        o_ref[...] = hh + jnp.maximum(t, 0.0)

    return pl.pallas_call(
        body,
        grid=(n // r,),
        in_specs=[
            pl.BlockSpec((2, r, d), lambda i: (0, i, 0)),
            pl.BlockSpec((2, r, 16), lambda i: (0, i, 0)),
            pl.BlockSpec((r, d), lambda i: (i, 0)),
            pl.BlockSpec((d, d), lambda i: (0, 0)),
            pl.BlockSpec((d, d), lambda i: (0, 0)),
            pl.BlockSpec((1, d), lambda i: (0, 0)),
            pl.BlockSpec((1, d), lambda i: (0, 0)),
            pl.BlockSpec((1, d), lambda i: (0, 0)),
        ],
        out_specs=pl.BlockSpec((r, d), lambda i: (i, 0)),
        out_shape=jax.ShapeDtypeStruct((n, d), jnp.float32),
    )(p, degp, h, wl, wr, bb, gg, be)


def kernel(node_features, edge_index, Wl, Wr, b, gamma, beta):
    n, d = node_features.shape
    e = edge_index.shape[1]
    l = Wl.shape[0]

    chunks = -(-(-(-e // (NW * CH))) // 8) * 8
    e_pad = NW * chunks * CH
    n_pad = -(-(n + 1) // (NSUB * 8)) * (NSUB * 8)

    src = edge_index[0].astype(jnp.int32)
    dst = edge_index[1].astype(jnp.int32)
    pad = e_pad - e
    pad_dst = n + (jnp.arange(pad, dtype=jnp.int32) % (n_pad - n))
    src_r = jnp.concatenate([src, jnp.zeros((pad,), jnp.int32)]).reshape(NW, chunks, CH)
    dst_r = jnp.concatenate([dst, pad_dst]).reshape(NW, chunks, CH)

    sc_agg = _make_sc_agg(n, d, chunks, n_pad)
    sc_deg = _make_sc_deg(n, d, chunks, n_pad)

    degp = sc_deg(dst_r)[:, :, :16]

    h = node_features
    for i in range(l):
        p = sc_agg(h, src_r, dst_r)
        h = _tc_update(h, p, degp, Wl[i], Wr[i],
                       b[i].reshape(1, d), gamma[i].reshape(1, d),
                       beta[i].reshape(1, d))
    return h


# speedup vs baseline: 1.0763x; 1.0331x over previous
import functools

import jax
import jax.numpy as jnp
from jax import lax
from jax.experimental import pallas as pl
from jax.experimental.pallas import tpu as pltpu
from jax.experimental.pallas import tpu_sc as plsc

NCORES = 2
NSUB = 16
NW = NCORES * NSUB
CH = 128
LN_EPS = 1e-5


def _row_chunks(total):
    out, off = [], 0
    while off < total:
        sz = min(CH, total - off)
        out.append((off, sz))
        off += sz
    return out


def _zero_vmem(ref, rows, d, value=0.0):
    @pl.loop(0, rows)
    def _(i):
        @pl.loop(0, d // 16)
        def _(k):
            ref[i, pl.ds(k * 16, 16)] = jnp.full((16,), value, jnp.float32)


def _make_sc_agg(n, d, chunks0, chunks1, n_pad):
    rows_per_tile = n_pad // NSUB
    assert rows_per_tile % 8 == 0 and chunks0 % 8 == 0 and chunks1 % 8 == 0

    out_type = jax.ShapeDtypeStruct((NCORES, n_pad, d), jnp.float32)
    scratch = [
        pltpu.VMEM_SHARED((n_pad, d), jnp.float32),
        pltpu.VMEM((8, CH), jnp.int32),
        pltpu.VMEM((8, CH), jnp.int32),
        pltpu.VMEM((CH, d), jnp.float32),
        pltpu.VMEM((CH, d), jnp.float32),
        pltpu.SemaphoreType.DMA,
        pltpu.SemaphoreType.DMA,
    ]
    mesh = plsc.VectorSubcoreMesh(core_axis_name="core", subcore_axis_name="subcore")

    def body(h_hbm, src_hbm, dst_hbm, agg_out,
             agg_sp, src_v, dst_v, rows0, rows1, sem0, sem1):
        c = lax.axis_index("core")
        s = lax.axis_index("subcore")
        w = c * NSUB + s
        rows = (rows0, rows1)
        sems = (sem0, sem1)
        zbase = s * rows_per_tile

        with jax.named_scope("agg_zero"):
            _zero_vmem(rows0, CH, d)
            for off, sz in _row_chunks(rows_per_tile):
                pltpu.sync_copy(rows0.at[pl.ds(0, sz)],
                                agg_sp.at[pl.ds(zbase + off, sz)])

            plsc.subcore_barrier()

        def gather(j_ref, b):
            pltpu.async_copy(h_hbm.at[j_ref.at[pl.ds(0, 64)]],
                             rows[b].at[pl.ds(0, 64)], sems[b])
            pltpu.async_copy(h_hbm.at[j_ref.at[pl.ds(64, 64)]],
                             rows[b].at[pl.ds(64, 64)], sems[b])

        def wait(b):
            pltpu.make_async_copy(h_hbm.at[pl.ds(0, CH)], rows[b], sems[b]).wait()

        nblk = jnp.where(c == 0, chunks0 // 8, chunks1 // 8)

        with jax.named_scope("agg_edges"):
            def blk(jb, carry):
                pltpu.sync_copy(src_hbm.at[w].at[pl.ds(jb * 8, 8)], src_v)
                pltpu.sync_copy(dst_hbm.at[w].at[pl.ds(jb * 8, 8)], dst_v)
                gather(src_v.at[0], 0)
                for t in range(4):
                    c0, c1 = 2 * t, 2 * t + 1
                    gather(src_v.at[c1], 1)
                    wait(0)
                    pltpu.sync_copy(rows0, agg_sp.at[dst_v.at[c0]], add=True)
                    if c1 + 1 < 8:
                        gather(src_v.at[c1 + 1], 0)
                    wait(1)
                    pltpu.sync_copy(rows1, agg_sp.at[dst_v.at[c1]], add=True)
                return carry

            lax.fori_loop(0, nblk, blk, 0)

            plsc.subcore_barrier()

        with jax.named_scope("agg_writeout"):
            for off, sz in _row_chunks(rows_per_tile):
                pltpu.sync_copy(agg_sp.at[pl.ds(zbase + off, sz)],
                                rows0.at[pl.ds(0, sz)])
                pltpu.sync_copy(rows0.at[pl.ds(0, sz)],
                                agg_out.at[c].at[pl.ds(zbase + off, sz)])

    return pl.kernel(body, out_type=out_type, mesh=mesh, scratch_types=scratch)


def _make_sc_deg(n, d, chunks0, chunks1, n_pad):
    rows_per_tile = n_pad // NSUB
    assert rows_per_tile % 8 == 0 and chunks0 % 8 == 0 and chunks1 % 8 == 0

    out_type = jax.ShapeDtypeStruct((NCORES, n_pad, d), jnp.float32)
    scratch = [
        pltpu.VMEM_SHARED((n_pad, d), jnp.float32),
        pltpu.VMEM((8, CH), jnp.int32),
        pltpu.VMEM((CH, d), jnp.float32),
    ]
    mesh = plsc.VectorSubcoreMesh(core_axis_name="core", subcore_axis_name="subcore")

    def body(dst_hbm, deg_out, deg_sp, dst_v, rows_v):
        c = lax.axis_index("core")
        s = lax.axis_index("subcore")
        w = c * NSUB + s
        zbase = s * rows_per_tile

        _zero_vmem(rows_v, CH, d)
        for off, sz in _row_chunks(rows_per_tile):
            pltpu.sync_copy(rows_v.at[pl.ds(0, sz)],
                            deg_sp.at[pl.ds(zbase + off, sz)])

        _zero_vmem(rows_v, CH, d, value=1.0)

        plsc.subcore_barrier()

        nblk = jnp.where(c == 0, chunks0 // 8, chunks1 // 8)

        def blk(jb, carry):
            pltpu.sync_copy(dst_hbm.at[w].at[pl.ds(jb * 8, 8)], dst_v)

            @pl.loop(0, 8)
            def _(j):
                pltpu.sync_copy(rows_v, deg_sp.at[dst_v.at[j]], add=True)
            return carry

        lax.fori_loop(0, nblk, blk, 0)

        plsc.subcore_barrier()

        for off, sz in _row_chunks(rows_per_tile):
            pltpu.sync_copy(deg_sp.at[pl.ds(zbase + off, sz)],
                            rows_v.at[pl.ds(0, sz)])
            pltpu.sync_copy(rows_v.at[pl.ds(0, sz)],
                            deg_out.at[c].at[pl.ds(zbase + off, sz)])

    return pl.kernel(body, out_type=out_type, mesh=mesh, scratch_types=scratch)


def _tc_update(h, p, degp, wl, wr, bb, gg, be):
    n, d = h.shape
    r = 1000
    assert n % r == 0

    def body(p_ref, dg_ref, h_ref, wl_ref, wr_ref, b_ref, g_ref, be_ref, o_ref):
        deg = dg_ref[0, :, 0:1] + dg_ref[1, :, 0:1]
        deg = jnp.maximum(deg, 1.0)
        agg = (p_ref[0] + p_ref[1]) / deg
        hh = h_ref[...]
        t = (jnp.dot(agg, wl_ref[...], preferred_element_type=jnp.float32,
                     precision=lax.Precision.HIGHEST)
             + jnp.dot(hh, wr_ref[...], preferred_element_type=jnp.float32,
                       precision=lax.Precision.HIGHEST)
             + b_ref[...])
        mu = jnp.mean(t, axis=-1, keepdims=True)
        var = jnp.mean((t - mu) ** 2, axis=-1, keepdims=True)
        t = (t - mu) * lax.rsqrt(var + LN_EPS) * g_ref[...] + be_ref[...]
        o_ref[...] = hh + jnp.maximum(t, 0.0)

    return pl.pallas_call(
        body,
        grid=(n // r,),
        in_specs=[
            pl.BlockSpec((2, r, d), lambda i: (0, i, 0)),
            pl.BlockSpec((2, r, 16), lambda i: (0, i, 0)),
            pl.BlockSpec((r, d), lambda i: (i, 0)),
            pl.BlockSpec((d, d), lambda i: (0, 0)),
            pl.BlockSpec((d, d), lambda i: (0, 0)),
            pl.BlockSpec((1, d), lambda i: (0, 0)),
            pl.BlockSpec((1, d), lambda i: (0, 0)),
            pl.BlockSpec((1, d), lambda i: (0, 0)),
        ],
        out_specs=pl.BlockSpec((r, d), lambda i: (i, 0)),
        out_shape=jax.ShapeDtypeStruct((n, d), jnp.float32),
    )(p, degp, h, wl, wr, bb, gg, be)


def kernel(node_features, edge_index, Wl, Wr, b, gamma, beta):
    n, d = node_features.shape
    e = edge_index.shape[1]
    l = Wl.shape[0]

    tot = -(-(-(-e // (NSUB * CH))) // 8) * 8
    chunks1 = max(8, int(round(tot * 0.2 / 8)) * 8)
    chunks0 = tot - chunks1
    cmax = max(chunks0, chunks1)
    e_pad = NSUB * (chunks0 + chunks1) * CH
    n_pad = -(-(n + 1) // (NSUB * 8)) * (NSUB * 8)

    src = edge_index[0].astype(jnp.int32)
    dst = edge_index[1].astype(jnp.int32)
    pad = e_pad - e
    pad_dst = n + (jnp.arange(pad, dtype=jnp.int32) % (n_pad - n))
    src_p = jnp.concatenate([src, jnp.zeros((pad,), jnp.int32)])
    dst_p = jnp.concatenate([dst, pad_dst])

    e0 = NSUB * chunks0 * CH
    ew1 = chunks1 * CH
    fill = cmax - chunks1
    src_pieces = [src_p[:e0]]
    dst_pieces = [dst_p[:e0]]
    for wv in range(NSUB):
        src_pieces.append(src_p[e0 + wv * ew1: e0 + (wv + 1) * ew1])
        dst_pieces.append(dst_p[e0 + wv * ew1: e0 + (wv + 1) * ew1])
        if fill:
            src_pieces.append(jnp.zeros((fill * CH,), jnp.int32))
            dst_pieces.append(jnp.full((fill * CH,), n, jnp.int32))
    src_r = jnp.concatenate(src_pieces).reshape(NW, cmax, CH)
    dst_r = jnp.concatenate(dst_pieces).reshape(NW, cmax, CH)

    sc_agg = _make_sc_agg(n, d, chunks0, chunks1, n_pad)
    sc_deg = _make_sc_deg(n, d, chunks0, chunks1, n_pad)

    degp = sc_deg(dst_r)[:, :, :16]

    h = node_features
    for i in range(l):
        p = sc_agg(h, src_r, dst_r)
        h = _tc_update(h, p, degp, Wl[i], Wr[i],
                       b[i].reshape(1, d), gamma[i].reshape(1, d),
                       beta[i].reshape(1, d))
    return h
